# Initial kernel scaffold; baseline (speedup 1.0000x reference)
import jax
import jax.numpy as jnp
from jax.experimental import pallas as pl
from jax.experimental.pallas import tpu as pltpu

B, H, U, D = 32, 32, 8, 128
RES = 128
CACHE_S = 2 * RES + 1


def _update_kernel(offs_ref, kc_ref, vc_ref, kn_ref, vn_ref, ko_ref, vo_ref):
    b = pl.program_id(0)
    off = offs_ref[b]
    ko_ref[...] = kc_ref[...]
    vo_ref[...] = vc_ref[...]
    ko_ref[0, :, pl.ds(off, U), :] = kn_ref[0]
    vo_ref[0, :, pl.ds(off, U), :] = vn_ref[0]


def kernel(k_cache_buf, v_cache_buf, k_new, v_new, cache_seqlens, qcache_seqlens):
    offs = cache_seqlens - qcache_seqlens
    grid_spec = pltpu.PrefetchScalarGridSpec(
        num_scalar_prefetch=1,
        grid=(B,),
        in_specs=[
            pl.BlockSpec((1, H, CACHE_S, D), lambda b, offs: (b, 0, 0, 0)),
            pl.BlockSpec((1, H, CACHE_S, D), lambda b, offs: (b, 0, 0, 0)),
            pl.BlockSpec((1, H, U, D), lambda b, offs: (b, 0, 0, 0)),
            pl.BlockSpec((1, H, U, D), lambda b, offs: (b, 0, 0, 0)),
        ],
        out_specs=[
            pl.BlockSpec((1, H, CACHE_S, D), lambda b, offs: (b, 0, 0, 0)),
            pl.BlockSpec((1, H, CACHE_S, D), lambda b, offs: (b, 0, 0, 0)),
        ],
    )
    k_out, v_out = pl.pallas_call(
        _update_kernel,
        grid_spec=grid_spec,
        out_shape=[
            jax.ShapeDtypeStruct((B, H, CACHE_S, D), k_cache_buf.dtype),
            jax.ShapeDtypeStruct((B, H, CACHE_S, D), v_cache_buf.dtype),
        ],
        compiler_params=pltpu.CompilerParams(
            dimension_semantics=("arbitrary",),
        ),
    )(offs, k_cache_buf, v_cache_buf, k_new, v_new)
    return (k_out, v_out)


# speedup vs baseline: 60.5741x; 60.5741x over previous
import jax
import jax.numpy as jnp
from jax.experimental import pallas as pl
from jax.experimental.pallas import tpu as pltpu

B, H, U, D = 32, 32, 8, 128
RES = 128
CACHE_S = 2 * RES + 1


def _update_kernel(offs_ref, kc_ref, vc_ref, kn_ref, vn_ref, ko_ref, vo_ref):
    b = pl.program_id(0)
    off = offs_ref[b]
    ko_ref[...] = kc_ref[...]
    vo_ref[...] = vc_ref[...]
    a = pl.multiple_of((off // U) * U, U)
    r = off - (off // U) * U
    j = jax.lax.broadcasted_iota(jnp.int32, (1, H, 2 * U, D), 2)
    sel = jnp.bitwise_and(j - r, U - 1)
    mask = (j >= r) & (j < r + U)

    def place(new_ref, cache_ref, out_ref):
        cand = jnp.zeros((1, H, 2 * U, D), dtype=jnp.float32)
        for jj in range(U):
            cand = jnp.where(
                sel == jj, new_ref[:, :, jj:jj + 1, :].astype(jnp.float32), cand)
        win = cache_ref[0, :, pl.ds(a, 2 * U), :].astype(jnp.float32)
        merged = jnp.where(mask, cand, win[None])
        out_ref[0, :, pl.ds(a, 2 * U), :] = merged[0].astype(out_ref.dtype)

    place(kn_ref, kc_ref, ko_ref)
    place(vn_ref, vc_ref, vo_ref)


def kernel(k_cache_buf, v_cache_buf, k_new, v_new, cache_seqlens, qcache_seqlens):
    offs = cache_seqlens - qcache_seqlens
    grid_spec = pltpu.PrefetchScalarGridSpec(
        num_scalar_prefetch=1,
        grid=(B,),
        in_specs=[
            pl.BlockSpec((1, H, CACHE_S, D), lambda b, offs: (b, 0, 0, 0)),
            pl.BlockSpec((1, H, CACHE_S, D), lambda b, offs: (b, 0, 0, 0)),
            pl.BlockSpec((1, H, U, D), lambda b, offs: (b, 0, 0, 0)),
            pl.BlockSpec((1, H, U, D), lambda b, offs: (b, 0, 0, 0)),
        ],
        out_specs=[
            pl.BlockSpec((1, H, CACHE_S, D), lambda b, offs: (b, 0, 0, 0)),
            pl.BlockSpec((1, H, CACHE_S, D), lambda b, offs: (b, 0, 0, 0)),
        ],
    )
    k_out, v_out = pl.pallas_call(
        _update_kernel,
        grid_spec=grid_spec,
        out_shape=[
            jax.ShapeDtypeStruct((B, H, CACHE_S, D), k_cache_buf.dtype),
            jax.ShapeDtypeStruct((B, H, CACHE_S, D), v_cache_buf.dtype),
        ],
        compiler_params=pltpu.CompilerParams(
            dimension_semantics=("arbitrary",),
        ),
    )(offs, k_cache_buf, v_cache_buf, k_new, v_new)
    return (k_out, v_out)
